# baseline (device time: 202808 ns/iter reference)
import functools

import numpy as np
import jax
import jax.numpy as jnp
from jax import lax
from jax.experimental import pallas as pl
from jax.experimental.pallas import tpu as pltpu

N_DEV = 4
SQ = 256
D_MODEL = 768
HEADS_PER_DEV = 4
DH = 64
SCALE = 0.125

_SERIALIZE = False


def _rope_tables():
    inv = 1.0 / (10000.0 ** (np.arange(0, DH, 2) / DH))
    pos = np.arange(SQ)[:, None] * inv[None, :]
    cos = np.repeat(np.cos(pos), 2, axis=-1).astype(np.float32)
    sin = np.repeat(np.sin(pos), 2, axis=-1).astype(np.float32)
    R = np.zeros((DH, DH), np.float32)
    for k in range(DH // 2):
        R[2 * k + 1, 2 * k] = -1.0
        R[2 * k, 2 * k + 1] = 1.0
    return cos, sin, R


def kernel(x, Wq, Wk, Wv, Wo):
    b_local = x.shape[0]

    wqkv = jnp.stack([Wq, Wk, Wv])
    wqkv = wqkv.reshape(3, D_MODEL, HEADS_PER_DEV, DH).transpose(0, 2, 1, 3)
    wo = Wo.reshape(HEADS_PER_DEV, DH, D_MODEL)

    cos_np, sin_np, R_np = _rope_tables()
    cos = jnp.asarray(cos_np)
    sin = jnp.asarray(sin_np)
    rot = jnp.asarray(R_np)

    def body(x_ref, wqkv_ref, wo_ref, cos_ref, sin_ref, rot_ref, out_ref,
             qkv_buf, wo_buf, qkv_ssem, qkv_rsem, wo_ssem, wo_rsem):
        my = lax.axis_index("i")
        left = lax.rem(my + N_DEV - 1, N_DEV)
        right = lax.rem(my + 1, N_DEV)

        barrier = pltpu.get_barrier_semaphore()
        for nbr in (left, right):
            pl.semaphore_signal(
                barrier, inc=1,
                device_id=(nbr,), device_id_type=pl.DeviceIdType.MESH,
            )
        pl.semaphore_wait(barrier, 2)

        qkv_buf[0] = wqkv_ref[...]
        wo_buf[0] = wo_ref[...]

        cos_t = cos_ref[...]
        sin_t = sin_ref[...]
        R = rot_ref[...]

        accs = [
            jnp.zeros((SQ, D_MODEL), jnp.float32) for _ in range(b_local)
        ]

        def compute_chunk(h):
            for b in range(b_local):
                xb = x_ref[b]
                for hd in range(HEADS_PER_DEV):
                    q = jnp.dot(xb, qkv_buf[h, 0, hd],
                                preferred_element_type=jnp.float32)
                    k = jnp.dot(xb, qkv_buf[h, 1, hd],
                                preferred_element_type=jnp.float32)
                    v = jnp.dot(xb, qkv_buf[h, 2, hd],
                                preferred_element_type=jnp.float32)
                    q = q * cos_t + jnp.dot(
                        q, R, preferred_element_type=jnp.float32) * sin_t
                    k = k * cos_t + jnp.dot(
                        k, R, preferred_element_type=jnp.float32) * sin_t
                    s = lax.dot_general(
                        q, k, (((1,), (1,)), ((), ())),
                        preferred_element_type=jnp.float32) * SCALE
                    s = s - jnp.max(s, axis=1, keepdims=True)
                    e = jnp.exp(s)
                    w = e / jnp.sum(e, axis=1, keepdims=True)
                    ctx = jnp.dot(w, v, preferred_element_type=jnp.float32)
                    accs[b] = accs[b] + jnp.dot(
                        ctx, wo_buf[h, hd],
                        preferred_element_type=jnp.float32)

        qkv_sends = []
        wo_sends = []
        for h in range(N_DEV):
            if h > 0:
                qkv_sends[h - 1].wait_recv()
                wo_sends[h - 1].wait_recv()
            if h < N_DEV - 1:
                sq = pltpu.make_async_remote_copy(
                    src_ref=qkv_buf.at[h],
                    dst_ref=qkv_buf.at[h + 1],
                    send_sem=qkv_ssem.at[h],
                    recv_sem=qkv_rsem.at[h + 1],
                    device_id=(right,),
                    device_id_type=pl.DeviceIdType.MESH,
                )
                sq.start()
                sw = pltpu.make_async_remote_copy(
                    src_ref=wo_buf.at[h],
                    dst_ref=wo_buf.at[h + 1],
                    send_sem=wo_ssem.at[h],
                    recv_sem=wo_rsem.at[h + 1],
                    device_id=(right,),
                    device_id_type=pl.DeviceIdType.MESH,
                )
                sw.start()
                qkv_sends.append(sq)
                wo_sends.append(sw)
                if _SERIALIZE:
                    sq.wait_send()
                    sw.wait_send()
            compute_chunk(h)

        for b in range(b_local):
            out_ref[b] = accs[b]

        if not _SERIALIZE:
            for s in qkv_sends + wo_sends:
                s.wait_send()

        @functools.partial(
            pl.run_scoped, second_barrier=pltpu.SemaphoreType.REGULAR)
        def _(second_barrier):
            for nbr in (left, right):
                pl.semaphore_signal(
                    second_barrier, inc=1,
                    device_id=(nbr,), device_id_type=pl.DeviceIdType.MESH,
                )
            pl.semaphore_wait(second_barrier, 2)

    vmem = functools.partial(pl.BlockSpec, memory_space=pltpu.VMEM)
    return pl.pallas_call(
        body,
        out_shape=jax.ShapeDtypeStruct((b_local, SQ, D_MODEL), jnp.float32),
        in_specs=[vmem()] * 6,
        out_specs=vmem(),
        scratch_shapes=[
            pltpu.VMEM((N_DEV, 3, HEADS_PER_DEV, D_MODEL, DH), jnp.float32),
            pltpu.VMEM((N_DEV, HEADS_PER_DEV, DH, D_MODEL), jnp.float32),
            pltpu.SemaphoreType.DMA((N_DEV,)),
            pltpu.SemaphoreType.DMA((N_DEV,)),
            pltpu.SemaphoreType.DMA((N_DEV,)),
            pltpu.SemaphoreType.DMA((N_DEV,)),
        ],
        compiler_params=pltpu.CompilerParams(collective_id=0),
    )(x, wqkv, wo, cos, sin, rot)
